# baseline (speedup 1.0000x reference)
import jax
import jax.numpy as jnp
from jax import lax
from jax.experimental import pallas as pl
from jax.experimental.pallas import tpu as pltpu, tpu_sc as plsc

B = 4096
D = 6400
NC = 2
NS = 16
NW = NC * NS
L = 16
ROWS_PER_W = B // NW
RBLK = 8
NBLK = ROWS_PER_W // RBLK
NJ = D // L


def _permute_body(x_hbm, perm_hbm, out_hbm, perm_v, in_v, out_v):
    wid = lax.axis_index("s") * NC + lax.axis_index("c")
    base = wid * ROWS_PER_W
    pltpu.sync_copy(perm_hbm, perm_v)

    def blk(i, carry):
        e0 = (base + i * RBLK) * D
        pltpu.sync_copy(x_hbm.at[pl.ds(e0, RBLK * D)], in_v)

        def jloop(j, c):
            col_idx = perm_v[pl.ds(j * L, L)]
            o = j * L
            for r in range(RBLK):
                vals = plsc.load_gather(in_v, [col_idx + r * D])
                out_v[pl.ds(o + r * D, L)] = vals
            return c

        lax.fori_loop(0, NJ, jloop, 0)
        pltpu.sync_copy(out_v, out_hbm.at[pl.ds(e0, RBLK * D)])
        return carry

    lax.fori_loop(0, NBLK, blk, 0)


@jax.jit
def _permute(inputs_flat, perm32):
    mesh = plsc.VectorSubcoreMesh(core_axis_name="c", subcore_axis_name="s")
    f = pl.kernel(
        _permute_body,
        out_type=jax.ShapeDtypeStruct((B * D,), jnp.float32),
        mesh=mesh,
        scratch_types=[
            pltpu.VMEM((D,), jnp.int32),
            pltpu.VMEM((RBLK * D,), jnp.float32),
            pltpu.VMEM((RBLK * D,), jnp.float32),
        ],
        compiler_params=pltpu.CompilerParams(
            needs_layout_passes=False,
            use_tc_tiling_on_sc=False,
        ),
    )
    return f(inputs_flat, perm32)


def kernel(inputs, perm):
    out_flat = _permute(inputs.reshape(-1), perm.astype(jnp.int32))
    return out_flat.reshape(B, D)


# speedup vs baseline: 1.1582x; 1.1582x over previous
import jax
import jax.numpy as jnp
from jax import lax
from jax.experimental import pallas as pl
from jax.experimental.pallas import tpu as pltpu, tpu_sc as plsc

B = 4096
D = 6400
NC = 2
NS = 16
NW = NC * NS
L = 16
ROWS_PER_W = B // NW
RBLK = 4
NBLK = ROWS_PER_W // RBLK
NJ = D // L
BLKE = RBLK * D


def _gather_block(perm_v, in_v, out_v):
    def jloop(j, c):
        col = perm_v[pl.ds(j * L, L)]
        o = j * L
        for r in range(RBLK):
            out_v[pl.ds(o + r * D, L)] = plsc.load_gather(in_v, [col + r * D])
        return c

    lax.fori_loop(0, NJ, jloop, 0)


def _permute_body(x_hbm, perm_hbm, out_hbm, perm_v,
                  in0, in1, out0, out1, si0, si1, so0, so1):
    wid = lax.axis_index("s") * NC + lax.axis_index("c")
    base_e = wid * ROWS_PER_W * D
    pltpu.sync_copy(perm_hbm, perm_v)

    def in_copy(blk, buf, sem):
        return pltpu.make_async_copy(
            x_hbm.at[pl.ds(base_e + blk * BLKE, BLKE)], buf, sem)

    def out_copy(blk, buf, sem):
        return pltpu.make_async_copy(
            buf, out_hbm.at[pl.ds(base_e + blk * BLKE, BLKE)], sem)

    in_copy(0, in0, si0).start()
    in_copy(1, in1, si1).start()
    in_copy(0, in0, si0).wait()
    _gather_block(perm_v, in0, out0)
    in_copy(2, in0, si0).start()
    out_copy(0, out0, so0).start()
    in_copy(1, in1, si1).wait()
    _gather_block(perm_v, in1, out1)
    in_copy(3, in1, si1).start()
    out_copy(1, out1, so1).start()

    def steady(t, c):
        b0 = 2 * t
        in_copy(b0, in0, si0).wait()
        out_copy(b0 - 2, out0, so0).wait()
        _gather_block(perm_v, in0, out0)
        in_copy(b0 + 2, in0, si0).start()
        out_copy(b0, out0, so0).start()
        in_copy(b0 + 1, in1, si1).wait()
        out_copy(b0 - 1, out1, so1).wait()
        _gather_block(perm_v, in1, out1)
        in_copy(b0 + 3, in1, si1).start()
        out_copy(b0 + 1, out1, so1).start()
        return c

    lax.fori_loop(1, NBLK // 2 - 1, steady, 0)

    bl = NBLK - 2
    in_copy(bl, in0, si0).wait()
    out_copy(bl - 2, out0, so0).wait()
    _gather_block(perm_v, in0, out0)
    out_copy(bl, out0, so0).start()
    in_copy(bl + 1, in1, si1).wait()
    out_copy(bl - 1, out1, so1).wait()
    _gather_block(perm_v, in1, out1)
    out_copy(bl + 1, out1, so1).start()
    out_copy(bl, out0, so0).wait()
    out_copy(bl + 1, out1, so1).wait()


@jax.jit
def _permute(inputs_flat, perm32):
    mesh = plsc.VectorSubcoreMesh(core_axis_name="c", subcore_axis_name="s")
    f = pl.kernel(
        _permute_body,
        out_type=jax.ShapeDtypeStruct((B * D,), jnp.float32),
        mesh=mesh,
        scratch_types=[
            pltpu.VMEM((D,), jnp.int32),
            pltpu.VMEM((BLKE,), jnp.float32),
            pltpu.VMEM((BLKE,), jnp.float32),
            pltpu.VMEM((BLKE,), jnp.float32),
            pltpu.VMEM((BLKE,), jnp.float32),
            pltpu.SemaphoreType.DMA,
            pltpu.SemaphoreType.DMA,
            pltpu.SemaphoreType.DMA,
            pltpu.SemaphoreType.DMA,
        ],
        compiler_params=pltpu.CompilerParams(
            needs_layout_passes=False,
            use_tc_tiling_on_sc=False,
        ),
    )
    return f(inputs_flat, perm32)


def kernel(inputs, perm):
    out_flat = _permute(inputs.reshape(-1), perm.astype(jnp.int32))
    return out_flat.reshape(B, D)


# speedup vs baseline: 1.7990x; 1.5533x over previous
import jax
import jax.numpy as jnp
from jax import lax
from jax.experimental import pallas as pl
from jax.experimental.pallas import tpu as pltpu, tpu_sc as plsc

B = 4096
D = 6400
NC = 2
NS = 16
NW = NC * NS
L = 16
ROWS_PER_W = B // NW
RBLK = 4
NBLK = ROWS_PER_W // RBLK
NJ = D // L
BLKE = RBLK * D


def _gather_block(perm_v, in_v, out_v):
    @plsc.parallel_loop(0, NJ, 1, unroll=8)
    def _(j):
        col = perm_v[pl.ds(j * L, L)]
        o = j * L
        for r in range(RBLK):
            out_v[pl.ds(o + r * D, L)] = plsc.load_gather(in_v, [col + r * D])


def _permute_body(x_hbm, perm_hbm, out_hbm, perm_v,
                  in0, in1, out0, out1, si0, si1, so0, so1):
    wid = lax.axis_index("s") * NC + lax.axis_index("c")
    base_e = wid * ROWS_PER_W * D
    pltpu.sync_copy(perm_hbm, perm_v)

    def in_copy(blk, buf, sem):
        return pltpu.make_async_copy(
            x_hbm.at[pl.ds(base_e + blk * BLKE, BLKE)], buf, sem)

    def out_copy(blk, buf, sem):
        return pltpu.make_async_copy(
            buf, out_hbm.at[pl.ds(base_e + blk * BLKE, BLKE)], sem)

    in_copy(0, in0, si0).start()
    in_copy(1, in1, si1).start()
    in_copy(0, in0, si0).wait()
    _gather_block(perm_v, in0, out0)
    in_copy(2, in0, si0).start()
    out_copy(0, out0, so0).start()
    in_copy(1, in1, si1).wait()
    _gather_block(perm_v, in1, out1)
    in_copy(3, in1, si1).start()
    out_copy(1, out1, so1).start()

    def steady(t, c):
        b0 = 2 * t
        in_copy(b0, in0, si0).wait()
        out_copy(b0 - 2, out0, so0).wait()
        _gather_block(perm_v, in0, out0)
        in_copy(b0 + 2, in0, si0).start()
        out_copy(b0, out0, so0).start()
        in_copy(b0 + 1, in1, si1).wait()
        out_copy(b0 - 1, out1, so1).wait()
        _gather_block(perm_v, in1, out1)
        in_copy(b0 + 3, in1, si1).start()
        out_copy(b0 + 1, out1, so1).start()
        return c

    lax.fori_loop(1, NBLK // 2 - 1, steady, 0)

    bl = NBLK - 2
    in_copy(bl, in0, si0).wait()
    out_copy(bl - 2, out0, so0).wait()
    _gather_block(perm_v, in0, out0)
    out_copy(bl, out0, so0).start()
    in_copy(bl + 1, in1, si1).wait()
    out_copy(bl - 1, out1, so1).wait()
    _gather_block(perm_v, in1, out1)
    out_copy(bl + 1, out1, so1).start()
    out_copy(bl, out0, so0).wait()
    out_copy(bl + 1, out1, so1).wait()


@jax.jit
def _permute(inputs_flat, perm32):
    mesh = plsc.VectorSubcoreMesh(core_axis_name="c", subcore_axis_name="s")
    f = pl.kernel(
        _permute_body,
        out_type=jax.ShapeDtypeStruct((B * D,), jnp.float32),
        mesh=mesh,
        scratch_types=[
            pltpu.VMEM((D,), jnp.int32),
            pltpu.VMEM((BLKE,), jnp.float32),
            pltpu.VMEM((BLKE,), jnp.float32),
            pltpu.VMEM((BLKE,), jnp.float32),
            pltpu.VMEM((BLKE,), jnp.float32),
            pltpu.SemaphoreType.DMA,
            pltpu.SemaphoreType.DMA,
            pltpu.SemaphoreType.DMA,
            pltpu.SemaphoreType.DMA,
        ],
        compiler_params=pltpu.CompilerParams(
            needs_layout_passes=False,
            use_tc_tiling_on_sc=False,
        ),
    )
    return f(inputs_flat, perm32)


def kernel(inputs, perm):
    out_flat = _permute(inputs.reshape(-1), perm.astype(jnp.int32))
    return out_flat.reshape(B, D)


# speedup vs baseline: 3.5119x; 1.9521x over previous
import jax
import jax.numpy as jnp
from jax import lax
from jax.experimental import pallas as pl
from jax.experimental.pallas import tpu as pltpu, tpu_sc as plsc

B = 4096
D = 6400
NC = 2
NS = 16
NW = NC * NS
L = 16
ROWS_PER_W = B // NW
RBLK = 8
NBLK = ROWS_PER_W // RBLK
NJ = D // L


def _gather_block(perm_v, in_v, out_v):
    @plsc.parallel_loop(0, NJ, 1, unroll=4)
    def _(j):
        col = perm_v[pl.ds(j * L, L)]
        o = j * L
        for r in range(RBLK):
            row = jnp.full((L,), r, jnp.int32)
            out_v[r, pl.ds(o, L)] = plsc.load_gather(in_v, [row, col])


def _permute_body(x_hbm, perm_hbm, out_hbm, perm_v, in_v, out_v):
    wid = lax.axis_index("s") * NC + lax.axis_index("c")
    base_r = wid * ROWS_PER_W
    pltpu.sync_copy(perm_hbm, perm_v)

    def blk(i, c):
        r0 = base_r + i * RBLK
        pltpu.sync_copy(x_hbm.at[pl.ds(r0, RBLK)], in_v)
        _gather_block(perm_v, in_v, out_v)
        pltpu.sync_copy(out_v, out_hbm.at[pl.ds(r0, RBLK)])
        return c

    lax.fori_loop(0, NBLK, blk, 0)


@jax.jit
def _permute(inputs, perm32):
    mesh = plsc.VectorSubcoreMesh(core_axis_name="c", subcore_axis_name="s")
    f = pl.kernel(
        _permute_body,
        out_type=jax.ShapeDtypeStruct((B, D), jnp.float32),
        mesh=mesh,
        scratch_types=[
            pltpu.VMEM((D,), jnp.int32),
            pltpu.VMEM((RBLK, D), jnp.float32),
            pltpu.VMEM((RBLK, D), jnp.float32),
        ],
        compiler_params=pltpu.CompilerParams(
            needs_layout_passes=False,
        ),
    )
    return f(inputs, perm32)


def kernel(inputs, perm):
    return _permute(inputs, perm.astype(jnp.int32))


# speedup vs baseline: 5.0423x; 1.4358x over previous
import jax
import jax.numpy as jnp
from jax import lax
from jax.experimental import pallas as pl
from jax.experimental.pallas import tpu as pltpu, tpu_sc as plsc

B = 4096
D = 6400
NC = 2
NS = 16
NW = NC * NS
L = 16
ROWS_PER_W = B // NW
RBLK = 8
NBLK = ROWS_PER_W // RBLK
CCH = 640
NCH = D // CCH
NJC = CCH // L


def _gather_chunk(k, perm_v, in_v, out_c):
    @plsc.parallel_loop(0, NJC, 1, unroll=4)
    def _(jj):
        col = perm_v[pl.ds(k * CCH + jj * L, L)]
        o = jj * L
        for r in range(RBLK):
            row = jnp.full((L,), r, jnp.int32)
            out_c[r, pl.ds(o, L)] = plsc.load_gather(in_v, [row, col])


def _permute_body(x_hbm, perm_hbm, out_hbm, perm_v,
                  in0, in1, oc0, oc1, si0, si1, so0, so1):
    wid = lax.axis_index("s") * NC + lax.axis_index("c")
    base_r = wid * ROWS_PER_W
    pltpu.sync_copy(perm_hbm, perm_v)

    ocs = (oc0, oc1)
    sos = (so0, so1)

    def in_copy(blk, buf, sem):
        return pltpu.make_async_copy(
            x_hbm.at[pl.ds(base_r + blk * RBLK, RBLK)], buf, sem)

    def out_copy(blk, k, buf, sem):
        return pltpu.make_async_copy(
            buf, out_hbm.at[pl.ds(base_r + blk * RBLK, RBLK),
                            pl.ds(k * CCH, CCH)], sem)

    def chunk_pair(blk, k0, in_v, wait_first):
        if wait_first:
            out_copy(blk, k0, oc0, so0).wait()
        _gather_chunk(k0, perm_v, in_v, oc0)
        out_copy(blk, k0, oc0, so0).start()
        if wait_first:
            out_copy(blk, k0 + 1, oc1, so1).wait()
        _gather_chunk(k0 + 1, perm_v, in_v, oc1)
        out_copy(blk, k0 + 1, oc1, so1).start()

    def do_block(blk, in_v):
        def pair(u, c):
            chunk_pair(blk, 2 * u, in_v, True)
            return c
        lax.fori_loop(0, NCH // 2, pair, 0)

    def do_block_first(blk, in_v):
        chunk_pair(blk, 0, in_v, wait_first=False)

        def pair(u, c):
            chunk_pair(blk, 2 * u, in_v, True)
            return c
        lax.fori_loop(1, NCH // 2, pair, 0)

    in_copy(0, in0, si0).start()
    in_copy(1, in1, si1).start()
    in_copy(0, in0, si0).wait()
    do_block_first(0, in0)
    in_copy(2, in0, si0).start()
    in_copy(1, in1, si1).wait()
    do_block(1, in1)
    in_copy(3, in1, si1).start()

    def steady(t, c):
        b0 = 2 * t
        in_copy(b0, in0, si0).wait()
        do_block(b0, in0)
        in_copy(b0 + 2, in0, si0).start()
        in_copy(b0 + 1, in1, si1).wait()
        do_block(b0 + 1, in1)
        in_copy(b0 + 3, in1, si1).start()
        return c

    lax.fori_loop(1, NBLK // 2 - 1, steady, 0)

    bl = NBLK - 2
    in_copy(bl, in0, si0).wait()
    do_block(bl, in0)
    in_copy(bl + 1, in1, si1).wait()
    do_block(bl + 1, in1)
    out_copy(bl + 1, NCH - 2, oc0, so0).wait()
    out_copy(bl + 1, NCH - 1, oc1, so1).wait()


@jax.jit
def _permute(inputs, perm32):
    mesh = plsc.VectorSubcoreMesh(core_axis_name="c", subcore_axis_name="s")
    f = pl.kernel(
        _permute_body,
        out_type=jax.ShapeDtypeStruct((B, D), jnp.float32),
        mesh=mesh,
        scratch_types=[
            pltpu.VMEM((D,), jnp.int32),
            pltpu.VMEM((RBLK, D), jnp.float32),
            pltpu.VMEM((RBLK, D), jnp.float32),
            pltpu.VMEM((RBLK, CCH), jnp.float32),
            pltpu.VMEM((RBLK, CCH), jnp.float32),
            pltpu.SemaphoreType.DMA,
            pltpu.SemaphoreType.DMA,
            pltpu.SemaphoreType.DMA,
            pltpu.SemaphoreType.DMA,
        ],
        compiler_params=pltpu.CompilerParams(
            needs_layout_passes=False,
        ),
    )
    return f(inputs, perm32)


def kernel(inputs, perm):
    return _permute(inputs, perm.astype(jnp.int32))
